# baseline (speedup 1.0000x reference)
import functools

import jax
import jax.numpy as jnp
from jax import lax
from jax.experimental import pallas as pl
from jax.experimental.pallas import tpu as pltpu
from jax.experimental.pallas import tpu_sc as plsc

_B, _N = 2, 8192
_NPOINTS = 2048
_AGGC = 128
_RADII = [0.2, 0.4, 0.8]
_NSAMPLES = [16, 32, 64]
_SUBL, _LANE = 64, 128
_OSUB = 16


def _fps_body(xr, inds_ref, nxyz_ref):
    X = xr[0, 0]
    Y = xr[0, 1]
    Z = xr[0, 2]
    r_io = jax.lax.broadcasted_iota(jnp.int32, (_SUBL, _LANE), 0)
    c_io = jax.lax.broadcasted_iota(jnp.int32, (_SUBL, _LANE), 1)
    fi = r_io * _LANE + c_io
    r2 = jax.lax.broadcasted_iota(jnp.int32, (_OSUB, _LANE), 0)
    c2 = jax.lax.broadcasted_iota(jnp.int32, (_OSUB, _LANE), 1)
    fo = r2 * _LANE + c2
    BIG = jnp.int32(1 << 30)

    def step(i, st):
        dist, far, idxs, nx, ny, nz = st
        sel = fi == far
        cx = jnp.sum(jnp.where(sel, X, 0.0))
        cy = jnp.sum(jnp.where(sel, Y, 0.0))
        cz = jnp.sum(jnp.where(sel, Z, 0.0))
        dx = X - cx
        dy = Y - cy
        dz = Z - cz
        d = (dx * dx + dy * dy) + dz * dz
        dist = jnp.minimum(dist, d)
        m = jnp.max(dist)
        far_new = jnp.min(jnp.where(dist == m, fi, BIG))
        w = fo == i
        idxs = jnp.where(w, far, idxs)
        nx = jnp.where(w, cx, nx)
        ny = jnp.where(w, cy, ny)
        nz = jnp.where(w, cz, nz)
        return (dist, far_new, idxs, nx, ny, nz)

    dist0 = jnp.full((_SUBL, _LANE), 1e10, jnp.float32)
    zi = jnp.zeros((_OSUB, _LANE), jnp.int32)
    zf = jnp.zeros((_OSUB, _LANE), jnp.float32)
    _, _, idxs, nx, ny, nz = jax.lax.fori_loop(
        0, _NPOINTS, step, (dist0, jnp.int32(0), zi, zf, zf, zf))
    inds_ref[0] = idxs
    nxyz_ref[0, 0] = nx
    nxyz_ref[0, 1] = ny
    nxyz_ref[0, 2] = nz


def _run_fps(xyz):
    xr = xyz.transpose(0, 2, 1).reshape(_B, 3, _SUBL, _LANE)
    inds, nxyz = pl.pallas_call(
        _fps_body,
        grid=(_B,),
        in_specs=[pl.BlockSpec((1, 3, _SUBL, _LANE), lambda b: (b, 0, 0, 0))],
        out_specs=[
            pl.BlockSpec((1, _OSUB, _LANE), lambda b: (b, 0, 0)),
            pl.BlockSpec((1, 3, _OSUB, _LANE), lambda b: (b, 0, 0, 0)),
        ],
        out_shape=[
            jax.ShapeDtypeStruct((_B, _OSUB, _LANE), jnp.int32),
            jax.ShapeDtypeStruct((_B, 3, _OSUB, _LANE), jnp.float32),
        ],
    )(xr)
    inds = inds.reshape(_B, _NPOINTS)
    new_xyz = nxyz.reshape(_B, 3, _NPOINTS).transpose(0, 2, 1)
    return inds, new_xyz


def _sq_body(nx_ref, x_ref, o_ref):
    nx = nx_ref[0]
    x = x_ref[0]
    s1 = jnp.sum(nx ** 2, -1)[:, None]
    s2 = jnp.sum(x ** 2, -1)[None, :]
    dot = jax.lax.dot_general(nx, x, (((1,), (1,)), ((), ())),
                              preferred_element_type=jnp.float32)
    o_ref[0] = s1 + s2 - 2.0 * dot


def _run_sq(new_xyz, xyz):
    return pl.pallas_call(
        _sq_body,
        grid=(_B, 8),
        in_specs=[pl.BlockSpec((1, 256, 3), lambda b, m: (b, m, 0)),
                  pl.BlockSpec((1, _N, 3), lambda b, m: (b, 0, 0))],
        out_specs=pl.BlockSpec((1, 256, _N), lambda b, m: (b, m, 0)),
        out_shape=jax.ShapeDtypeStruct((_B, _NPOINTS, _N), jnp.float32),
    )(new_xyz, xyz)


_NCORE, _NSUB = 2, 16
_NW = _NCORE * _NSUB
_ROWS = _B * _NPOINTS
_RPW = _ROWS // _NW
_GRP = 16
_NGRP = _RPW // _GRP
_TAB_W = _N * 7
_TOT = [_ROWS * ns for ns in _NSAMPLES]
_RAD2 = [r * r for r in _RADII]


def _sc_body(tab_hbm, cen_hbm, sq_hbm, g1_hbm, g2_hbm, g3_hbm,
             tab_v, cen_v, sq_v, gb1, gb2, gb3, go1, go2, go3, cnt_s):
    wid = lax.axis_index("c") * _NSUB + lax.axis_index("s")
    base_row = wid * _RPW
    b = base_row // _NPOINTS
    pltpu.sync_copy(tab_hbm.at[pl.ds(b * _TAB_W, _TAB_W)], tab_v)
    pltpu.sync_copy(cen_hbm.at[pl.ds(base_row * 8, _RPW * 8)],
                    cen_v.at[pl.ds(0, _RPW * 8)])
    iota = lax.iota(jnp.int32, 16)
    zeros16 = jnp.zeros((16,), jnp.int32)
    gbs = (gb1, gb2, gb3)
    gos = (go1, go2, go3)
    ghs = (g1_hbm, g2_hbm, g3_hbm)

    def group_body(gr, carry):
        def row_body(rr, carry2):
            rloc = gr * _GRP + rr
            cvec = cen_v[pl.ds(rloc * 8, 16)]
            cx = cvec[0]
            cy = cvec[1]
            cz = cvec[2]
            pltpu.sync_copy(
                sq_hbm.at[pl.ds((base_row + rloc) * _N, _N)], sq_v)
            for k in range(3):
                gbs[k][pl.ds(0, 16)] = zeros16
                cnt_s[k] = jnp.int32(0)

            def chunk_body(j, carry3):
                c1 = cnt_s[0]
                c2 = cnt_s[1]
                c3 = cnt_s[2]
                live = (c1 < 16) | (c2 < 32) | (c3 < 64)

                @pl.when(live)
                def _do():
                    idx = j * 16 + iota
                    sq = sq_v[pl.ds(j * 16, 16)]
                    cs = [c1, c2, c3]
                    for k in range(3):
                        mk = sq <= _RAD2[k]
                        csum = plsc.cumsum(mk.astype(jnp.int32))
                        pos = (cs[k] - 1) + csum
                        plsc.store_scatter(gbs[k], [pos], idx, mask=mk)
                        cnt_s[k] = jnp.minimum(
                            cs[k] + jnp.max(csum), _NSAMPLES[k])
                return carry3

            lax.fori_loop(0, _N // 16, chunk_body, 0)
            cnts = (cnt_s[0], cnt_s[1], cnt_s[2])
            for k in range(3):
                ns = _NSAMPLES[k]
                first = plsc.load_gather(gbs[k], [zeros16])
                for t in range(ns // 16):
                    lane = t * 16 + iota
                    v = gbs[k][pl.ds(t * 16, 16)]
                    v = jnp.where(lane < cnts[k], v, first)
                    v7 = v * 7
                    colb = rr * ns + t * 16
                    for c in range(7):
                        val = plsc.load_gather(tab_v, [v7 + c])
                        if c == 0:
                            val = val - cx
                        elif c == 1:
                            val = val - cy
                        elif c == 2:
                            val = val - cz
                        gos[k][pl.ds(c * (_GRP * ns) + colb, 16)] = val
            return carry2
        lax.fori_loop(0, _GRP, row_body, 0)
        for k in range(3):
            gsz = _GRP * _NSAMPLES[k]
            colbase = (base_row + gr * _GRP) * _NSAMPLES[k]
            for c in range(7):
                pltpu.sync_copy(
                    gos[k].at[pl.ds(c * gsz, gsz)],
                    ghs[k].at[pl.ds(c * _TOT[k] + colbase, gsz)])
        return carry
    lax.fori_loop(0, _NGRP, group_body, 0)


_sc_grouper = functools.partial(
    pl.kernel,
    mesh=plsc.VectorSubcoreMesh(core_axis_name="c", subcore_axis_name="s"),
    compiler_params=pltpu.CompilerParams(needs_layout_passes=False),
    out_type=[
        jax.ShapeDtypeStruct((7 * _TOT[0],), jnp.float32),
        jax.ShapeDtypeStruct((7 * _TOT[1],), jnp.float32),
        jax.ShapeDtypeStruct((7 * _TOT[2],), jnp.float32),
    ],
    scratch_types=[
        pltpu.VMEM((_TAB_W,), jnp.float32),
        pltpu.VMEM((_RPW * 8 + 8,), jnp.float32),
        pltpu.VMEM((_N,), jnp.float32),
        pltpu.VMEM((16 + 16,), jnp.int32),
        pltpu.VMEM((32 + 16,), jnp.int32),
        pltpu.VMEM((64 + 16,), jnp.int32),
        pltpu.VMEM((7 * _GRP * 16,), jnp.float32),
        pltpu.VMEM((7 * _GRP * 32,), jnp.float32),
        pltpu.VMEM((7 * _GRP * 64,), jnp.float32),
        pltpu.SMEM((8,), jnp.int32),
    ],
)(_sc_body)


def _run_grouper(xyz, feature, new_xyz):
    feat_t = jnp.transpose(feature, (0, 2, 1))
    tab = jnp.concatenate([xyz, feat_t], axis=-1).reshape(-1)
    cen8 = jnp.zeros((_B * _NPOINTS, 8), jnp.float32)
    cen8 = cen8.at[:, :3].set(new_xyz.reshape(_B * _NPOINTS, 3))
    cen = cen8.reshape(-1)
    sq = _run_sq(new_xyz, xyz).reshape(-1)
    g1, g2, g3 = _sc_grouper(tab, cen, sq)
    return [g.reshape(7, tot) for g, tot in zip((g1, g2, g3), _TOT)]


_TILE = 8192


def _mm_stats_body(nsteps, W_ref, b_ref, x_ref, y_ref, st_ref, acc_ref):
    step = pl.program_id(0)
    y = jax.lax.dot_general(W_ref[...], x_ref[...], (((1,), (0,)), ((), ())),
                            preferred_element_type=jnp.float32) + b_ref[...]
    y_ref[...] = y
    c = y.shape[0]
    ys = y.reshape(c, y.shape[1] // 128, 128)
    s = jnp.sum(ys, axis=1)
    s2 = jnp.sum(ys * ys, axis=1)

    @pl.when(step == 0)
    def _init():
        acc_ref[0] = s
        acc_ref[1] = s2

    @pl.when(step > 0)
    def _acc():
        acc_ref[0] += s
        acc_ref[1] += s2

    @pl.when(step == nsteps - 1)
    def _emit():
        st_ref[...] = acc_ref[...]


def _bn_relu(y, mu_ref, iv_ref, gm_ref, bt_ref):
    xh = (y - mu_ref[...]) * iv_ref[...]
    return jax.nn.relu(xh * gm_ref[...] + bt_ref[...])


def _bn_mm_stats_body(nsteps, mu_ref, iv_ref, gm_ref, bt_ref, W_ref, b_ref,
                      x_ref, y_ref, st_ref, acc_ref):
    step = pl.program_id(0)
    h = _bn_relu(x_ref[...], mu_ref, iv_ref, gm_ref, bt_ref)
    y = jax.lax.dot_general(W_ref[...], h, (((1,), (0,)), ((), ())),
                            preferred_element_type=jnp.float32) + b_ref[...]
    y_ref[...] = y
    c = y.shape[0]
    ys = y.reshape(c, y.shape[1] // 128, 128)
    s = jnp.sum(ys, axis=1)
    s2 = jnp.sum(ys * ys, axis=1)

    @pl.when(step == 0)
    def _init():
        acc_ref[0] = s
        acc_ref[1] = s2

    @pl.when(step > 0)
    def _acc():
        acc_ref[0] += s
        acc_ref[1] += s2

    @pl.when(step == nsteps - 1)
    def _emit():
        st_ref[...] = acc_ref[...]


def _bn_max_body(ns, mu_ref, iv_ref, gm_ref, bt_ref, x_ref, o_ref):
    h = _bn_relu(x_ref[...], mu_ref, iv_ref, gm_ref, bt_ref)
    c, t = h.shape
    o_ref[...] = jnp.max(h.reshape(c, t // ns, ns), axis=-1)


def _bn_relu_body(mu_ref, iv_ref, gm_ref, bt_ref, x_ref, o_ref):
    o_ref[...] = _bn_relu(x_ref[...], mu_ref, iv_ref, gm_ref, bt_ref)


def _col2(v):
    return v.reshape(-1, 1)


def _vec_spec(c):
    return pl.BlockSpec((c, 1), lambda t: (0, 0))


def _mm_stats(W, b, x):
    cout, cin = W.shape
    p = x.shape[1]
    tile = min(_TILE, p)
    nsteps = p // tile
    y, st = pl.pallas_call(
        functools.partial(_mm_stats_body, nsteps),
        grid=(nsteps,),
        in_specs=[pl.BlockSpec((cout, cin), lambda t: (0, 0)),
                  _vec_spec(cout),
                  pl.BlockSpec((cin, tile), lambda t: (0, t))],
        out_specs=[pl.BlockSpec((cout, tile), lambda t: (0, t)),
                   pl.BlockSpec((2, cout, 128), lambda t: (0, 0, 0))],
        out_shape=[jax.ShapeDtypeStruct((cout, p), jnp.float32),
                   jax.ShapeDtypeStruct((2, cout, 128), jnp.float32)],
        scratch_shapes=[pltpu.VMEM((2, cout, 128), jnp.float32)],
    )(W, _col2(b), x)
    return y, st


def _bn_mm_stats(mu, iv, gm, bt, W, b, x):
    cout, cin = W.shape
    p = x.shape[1]
    nsteps = p // _TILE
    y, st = pl.pallas_call(
        functools.partial(_bn_mm_stats_body, nsteps),
        grid=(nsteps,),
        in_specs=[_vec_spec(cin), _vec_spec(cin), _vec_spec(cin),
                  _vec_spec(cin),
                  pl.BlockSpec((cout, cin), lambda t: (0, 0)),
                  _vec_spec(cout),
                  pl.BlockSpec((cin, _TILE), lambda t: (0, t))],
        out_specs=[pl.BlockSpec((cout, _TILE), lambda t: (0, t)),
                   pl.BlockSpec((2, cout, 128), lambda t: (0, 0, 0))],
        out_shape=[jax.ShapeDtypeStruct((cout, p), jnp.float32),
                   jax.ShapeDtypeStruct((2, cout, 128), jnp.float32)],
        scratch_shapes=[pltpu.VMEM((2, cout, 128), jnp.float32)],
    )(_col2(mu), _col2(iv), _col2(gm), _col2(bt), W, _col2(b), x)
    return y, st


def _bn_max(mu, iv, gm, bt, x, ns):
    c, p = x.shape
    nsteps = p // _TILE
    return pl.pallas_call(
        functools.partial(_bn_max_body, ns),
        grid=(nsteps,),
        in_specs=[_vec_spec(c), _vec_spec(c), _vec_spec(c), _vec_spec(c),
                  pl.BlockSpec((c, _TILE), lambda t: (0, t))],
        out_specs=pl.BlockSpec((c, _TILE // ns), lambda t: (0, t)),
        out_shape=jax.ShapeDtypeStruct((c, p // ns), jnp.float32),
    )(_col2(mu), _col2(iv), _col2(gm), _col2(bt), x)


def _bn_relu_call(mu, iv, gm, bt, x):
    c, p = x.shape
    return pl.pallas_call(
        _bn_relu_body,
        grid=(1,),
        in_specs=[_vec_spec(c), _vec_spec(c), _vec_spec(c), _vec_spec(c),
                  pl.BlockSpec((c, p), lambda t: (0, 0))],
        out_specs=pl.BlockSpec((c, p), lambda t: (0, 0)),
        out_shape=jax.ShapeDtypeStruct((c, p), jnp.float32),
    )(_col2(mu), _col2(iv), _col2(gm), _col2(bt), x)


def _finalize_stats(st, p):
    s = st[0].sum(-1)
    s2 = st[1].sum(-1)
    mean = s / p
    var = s2 / p - mean * mean
    return mean, jax.lax.rsqrt(var + 1e-5)


def _square_distance(src, dst):
    return (jnp.sum(src ** 2, -1)[:, :, None] + jnp.sum(dst ** 2, -1)[:, None, :]
            - 2.0 * jnp.einsum('bmd,bnd->bmn', src, dst))


def _ball_query(radius, nsample, xyz, new_xyz):
    b, n, _ = xyz.shape
    m = new_xyz.shape[1]
    sqr = _square_distance(new_xyz, xyz)
    gidx = jnp.broadcast_to(jnp.arange(n, dtype=jnp.int32), (b, m, n))
    gidx = jnp.where(sqr > radius ** 2, n, gidx)
    gidx = jnp.sort(gidx, axis=-1)[:, :, :nsample]
    first = gidx[:, :, :1]
    first = jnp.where(first == n, 0, first)
    gidx = jnp.where(gidx == n, first, gidx)
    return gidx


def _gather_points(points, idx):
    bsz = points.shape[0]
    bidx = jnp.arange(bsz).reshape((bsz,) + (1,) * (idx.ndim - 1))
    return points[bidx, idx]


def _batchnorm(x, gamma, beta, axes):
    mean = jnp.mean(x, axis=axes, keepdims=True)
    var = jnp.var(x, axis=axes, keepdims=True)
    xh = (x - mean) * jax.lax.rsqrt(var + 1e-5)
    shape = [1] * x.ndim
    shape[1] = x.shape[1]
    return xh * gamma.reshape(shape) + beta.reshape(shape)


def kernel(xyz, feature, mlp_params, conv1_W, conv1_b, bn1_gamma, bn1_beta, fps_idx):
    inds, new_xyz = _run_fps(xyz)
    xs = _run_grouper(xyz, feature, new_xyz)
    outs = []
    for i in range(len(_RADII)):
        x = xs[i]
        p = x.shape[1]
        ns = _NSAMPLES[i]
        mu = iv = None
        for li, (W, bb, gm, bt) in enumerate(mlp_params[i]):
            if li == 0:
                x, st = _mm_stats(W, bb, x)
            else:
                x, st = _bn_mm_stats(mu, iv, gm_prev, bt_prev, W, bb, x)
            mu, iv = _finalize_stats(st, p)
            gm_prev, bt_prev = gm, bt
        outs.append(_bn_max(mu, iv, gm_prev, bt_prev, x, ns))
    nf_in = jnp.concatenate(outs, axis=0)
    y, st = _mm_stats(conv1_W, conv1_b, nf_in)
    mu, iv = _finalize_stats(st, _B * _NPOINTS)
    nf = _bn_relu_call(mu, iv, bn1_gamma, bn1_beta, y)
    nf = nf.reshape(_AGGC, _B, _NPOINTS).transpose(1, 0, 2)
    return new_xyz, nf, inds


# speedup vs baseline: 1.0602x; 1.0602x over previous
import functools

import jax
import jax.numpy as jnp
from jax import lax
from jax.experimental import pallas as pl
from jax.experimental.pallas import tpu as pltpu
from jax.experimental.pallas import tpu_sc as plsc

_B, _N = 2, 8192
_NPOINTS = 2048
_AGGC = 128
_RADII = [0.2, 0.4, 0.8]
_NSAMPLES = [16, 32, 64]
_SUBL, _LANE = 64, 128
_OSUB = 16


def _fps_body(xr, inds_ref, nxyz_ref):
    X = xr[0, 0]
    Y = xr[0, 1]
    Z = xr[0, 2]
    r_io = jax.lax.broadcasted_iota(jnp.int32, (_SUBL, _LANE), 0)
    c_io = jax.lax.broadcasted_iota(jnp.int32, (_SUBL, _LANE), 1)
    fi = r_io * _LANE + c_io
    r2 = jax.lax.broadcasted_iota(jnp.int32, (_OSUB, _LANE), 0)
    c2 = jax.lax.broadcasted_iota(jnp.int32, (_OSUB, _LANE), 1)
    fo = r2 * _LANE + c2
    BIG = jnp.int32(1 << 30)

    def step(i, st):
        dist, far, idxs, nx, ny, nz = st
        sel = fi == far
        cx = jnp.sum(jnp.where(sel, X, 0.0))
        cy = jnp.sum(jnp.where(sel, Y, 0.0))
        cz = jnp.sum(jnp.where(sel, Z, 0.0))
        dx = X - cx
        dy = Y - cy
        dz = Z - cz
        d = (dx * dx + dy * dy) + dz * dz
        dist = jnp.minimum(dist, d)
        m = jnp.max(dist)
        far_new = jnp.min(jnp.where(dist == m, fi, BIG))
        w = fo == i
        idxs = jnp.where(w, far, idxs)
        nx = jnp.where(w, cx, nx)
        ny = jnp.where(w, cy, ny)
        nz = jnp.where(w, cz, nz)
        return (dist, far_new, idxs, nx, ny, nz)

    dist0 = jnp.full((_SUBL, _LANE), 1e10, jnp.float32)
    zi = jnp.zeros((_OSUB, _LANE), jnp.int32)
    zf = jnp.zeros((_OSUB, _LANE), jnp.float32)
    _, _, idxs, nx, ny, nz = jax.lax.fori_loop(
        0, _NPOINTS, step, (dist0, jnp.int32(0), zi, zf, zf, zf))
    inds_ref[0] = idxs
    nxyz_ref[0, 0] = nx
    nxyz_ref[0, 1] = ny
    nxyz_ref[0, 2] = nz


def _run_fps(xyz):
    xr = xyz.transpose(0, 2, 1).reshape(_B, 3, _SUBL, _LANE)
    inds, nxyz = pl.pallas_call(
        _fps_body,
        grid=(_B,),
        in_specs=[pl.BlockSpec((1, 3, _SUBL, _LANE), lambda b: (b, 0, 0, 0))],
        out_specs=[
            pl.BlockSpec((1, _OSUB, _LANE), lambda b: (b, 0, 0)),
            pl.BlockSpec((1, 3, _OSUB, _LANE), lambda b: (b, 0, 0, 0)),
        ],
        out_shape=[
            jax.ShapeDtypeStruct((_B, _OSUB, _LANE), jnp.int32),
            jax.ShapeDtypeStruct((_B, 3, _OSUB, _LANE), jnp.float32),
        ],
    )(xr)
    inds = inds.reshape(_B, _NPOINTS)
    new_xyz = nxyz.reshape(_B, 3, _NPOINTS).transpose(0, 2, 1)
    return inds, new_xyz


def _sq_body(nx_ref, x_ref, o_ref):
    nx = nx_ref[0]
    x = x_ref[0]
    s1 = jnp.sum(nx ** 2, -1)[:, None]
    s2 = jnp.sum(x ** 2, -1)[None, :]
    dot = jax.lax.dot_general(nx, x, (((1,), (1,)), ((), ())),
                              preferred_element_type=jnp.float32)
    o_ref[0] = s1 + s2 - 2.0 * dot


def _run_sq(new_xyz, xyz):
    return pl.pallas_call(
        _sq_body,
        grid=(_B, 8),
        in_specs=[pl.BlockSpec((1, 256, 3), lambda b, m: (b, m, 0)),
                  pl.BlockSpec((1, _N, 3), lambda b, m: (b, 0, 0))],
        out_specs=pl.BlockSpec((1, 256, _N), lambda b, m: (b, m, 0)),
        out_shape=jax.ShapeDtypeStruct((_B, _NPOINTS, _N), jnp.float32),
    )(new_xyz, xyz)


_NCORE, _NSUB = 2, 16
_NW = _NCORE * _NSUB
_ROWS = _B * _NPOINTS
_RPW = _ROWS // _NW
_GRP = 16
_NGRP = _RPW // _GRP
_TAB_W = _N * 7
_TOT = [_ROWS * ns for ns in _NSAMPLES]
_RAD2 = [r * r for r in _RADII]


def _sc_body(tab_hbm, cen_hbm, sq_hbm, g1_hbm, g2_hbm, g3_hbm,
             tab_v, cen_v, sq_v, gb1, gb2, gb3, go1, go2, go3, cnt_s,
             sem0, sem1):
    wid = lax.axis_index("c") * _NSUB + lax.axis_index("s")
    base_row = wid * _RPW
    b = base_row // _NPOINTS
    pltpu.sync_copy(tab_hbm.at[pl.ds(b * _TAB_W, _TAB_W)], tab_v)
    pltpu.sync_copy(cen_hbm.at[pl.ds(base_row * 8, _RPW * 8)],
                    cen_v.at[pl.ds(0, _RPW * 8)])
    iota = lax.iota(jnp.int32, 16)
    zeros16 = jnp.zeros((16,), jnp.int32)
    gbs = (gb1, gb2, gb3)
    gos = (go1, go2, go3)
    ghs = (g1_hbm, g2_hbm, g3_hbm)
    sems = (sem0, sem1)
    sq_base = base_row * _N
    npairs = _RPW // 2

    def scan_row(rr, rloc, soff):
        cvec = cen_v[pl.ds(rloc * 8, 16)]
        cx = cvec[0]
        cy = cvec[1]
        cz = cvec[2]
        for k in range(3):
            gbs[k][pl.ds(0, 16)] = zeros16
            cnt_s[k] = jnp.int32(0)

        def chunk_body(j, carry3):
            c1 = cnt_s[0]
            c2 = cnt_s[1]
            c3 = cnt_s[2]
            live = (c1 < 16) | (c2 < 32) | (c3 < 64)

            @pl.when(live)
            def _do():
                idx = j * 16 + iota
                sq = sq_v[pl.ds(soff + j * 16, 16)]
                cs = [c1, c2, c3]
                for k in range(3):
                    mk = sq <= _RAD2[k]
                    csum = plsc.cumsum(mk.astype(jnp.int32))
                    pos = (cs[k] - 1) + csum
                    plsc.store_scatter(gbs[k], [pos], idx, mask=mk)
                    cnt_s[k] = jnp.minimum(
                        cs[k] + jnp.max(csum), _NSAMPLES[k])
            return carry3

        lax.fori_loop(0, _N // 16, chunk_body, 0)
        cnts = (cnt_s[0], cnt_s[1], cnt_s[2])
        for k in range(3):
            ns = _NSAMPLES[k]
            first = plsc.load_gather(gbs[k], [zeros16])
            for t in range(ns // 16):
                lane = t * 16 + iota
                v = gbs[k][pl.ds(t * 16, 16)]
                v = jnp.where(lane < cnts[k], v, first)
                v7 = v * 7
                colb = rr * ns + t * 16
                for c in range(7):
                    val = plsc.load_gather(tab_v, [v7 + c])
                    if c == 0:
                        val = val - cx
                    elif c == 1:
                        val = val - cy
                    elif c == 2:
                        val = val - cz
                    gos[k][pl.ds(c * (_GRP * ns) + colb, 16)] = val

    pltpu.async_copy(sq_hbm.at[pl.ds(sq_base, 2 * _N)],
                     sq_v.at[pl.ds(0, 2 * _N)], sem0)

    def group_body(gr, carry):
        def pair_body(pp, carry2):
            gp = gr * (_GRP // 2) + pp

            def do_pair(par):
                pltpu.make_async_copy(
                    sq_hbm.at[pl.ds(sq_base + gp * 2 * _N, 2 * _N)],
                    sq_v.at[pl.ds(par * 2 * _N, 2 * _N)],
                    sems[par]).wait()

                @pl.when(gp + 1 < npairs)
                def _pref():
                    pltpu.async_copy(
                        sq_hbm.at[pl.ds(sq_base + (gp + 1) * 2 * _N, 2 * _N)],
                        sq_v.at[pl.ds((1 - par) * 2 * _N, 2 * _N)],
                        sems[1 - par])
                for r01 in range(2):
                    rr = pp * 2 + r01
                    scan_row(rr, gr * _GRP + rr, (par * 2 + r01) * _N)

            @pl.when(pp % 2 == 0)
            def _even():
                do_pair(0)

            @pl.when(pp % 2 == 1)
            def _odd():
                do_pair(1)
            return carry2
        lax.fori_loop(0, _GRP // 2, pair_body, 0)
        for k in range(3):
            gsz = _GRP * _NSAMPLES[k]
            colbase = (base_row + gr * _GRP) * _NSAMPLES[k]
            for c in range(7):
                pltpu.sync_copy(
                    gos[k].at[pl.ds(c * gsz, gsz)],
                    ghs[k].at[pl.ds(c * _TOT[k] + colbase, gsz)])
        return carry
    lax.fori_loop(0, _NGRP, group_body, 0)


_sc_grouper = functools.partial(
    pl.kernel,
    mesh=plsc.VectorSubcoreMesh(core_axis_name="c", subcore_axis_name="s"),
    compiler_params=pltpu.CompilerParams(needs_layout_passes=False),
    out_type=[
        jax.ShapeDtypeStruct((7 * _TOT[0],), jnp.float32),
        jax.ShapeDtypeStruct((7 * _TOT[1],), jnp.float32),
        jax.ShapeDtypeStruct((7 * _TOT[2],), jnp.float32),
    ],
    scratch_types=[
        pltpu.VMEM((_TAB_W,), jnp.float32),
        pltpu.VMEM((_RPW * 8 + 8,), jnp.float32),
        pltpu.VMEM((4 * _N,), jnp.float32),
        pltpu.VMEM((16 + 16,), jnp.int32),
        pltpu.VMEM((32 + 16,), jnp.int32),
        pltpu.VMEM((64 + 16,), jnp.int32),
        pltpu.VMEM((7 * _GRP * 16,), jnp.float32),
        pltpu.VMEM((7 * _GRP * 32,), jnp.float32),
        pltpu.VMEM((7 * _GRP * 64,), jnp.float32),
        pltpu.SMEM((8,), jnp.int32),
        pltpu.SemaphoreType.DMA,
        pltpu.SemaphoreType.DMA,
    ],
)(_sc_body)


def _run_grouper(xyz, feature, new_xyz):
    feat_t = jnp.transpose(feature, (0, 2, 1))
    tab = jnp.concatenate([xyz, feat_t], axis=-1).reshape(-1)
    cen8 = jnp.zeros((_B * _NPOINTS, 8), jnp.float32)
    cen8 = cen8.at[:, :3].set(new_xyz.reshape(_B * _NPOINTS, 3))
    cen = cen8.reshape(-1)
    sq = _run_sq(new_xyz, xyz).reshape(-1)
    g1, g2, g3 = _sc_grouper(tab, cen, sq)
    return [g.reshape(7, tot) for g, tot in zip((g1, g2, g3), _TOT)]


_TILE = 8192


def _mm_stats_body(nsteps, W_ref, b_ref, x_ref, y_ref, st_ref, acc_ref):
    step = pl.program_id(0)
    y = jax.lax.dot_general(W_ref[...], x_ref[...], (((1,), (0,)), ((), ())),
                            preferred_element_type=jnp.float32) + b_ref[...]
    y_ref[...] = y
    c = y.shape[0]
    ys = y.reshape(c, y.shape[1] // 128, 128)
    s = jnp.sum(ys, axis=1)
    s2 = jnp.sum(ys * ys, axis=1)

    @pl.when(step == 0)
    def _init():
        acc_ref[0] = s
        acc_ref[1] = s2

    @pl.when(step > 0)
    def _acc():
        acc_ref[0] += s
        acc_ref[1] += s2

    @pl.when(step == nsteps - 1)
    def _emit():
        st_ref[...] = acc_ref[...]


def _bn_relu(y, mu_ref, iv_ref, gm_ref, bt_ref):
    xh = (y - mu_ref[...]) * iv_ref[...]
    return jax.nn.relu(xh * gm_ref[...] + bt_ref[...])


def _bn_mm_stats_body(nsteps, mu_ref, iv_ref, gm_ref, bt_ref, W_ref, b_ref,
                      x_ref, y_ref, st_ref, acc_ref):
    step = pl.program_id(0)
    h = _bn_relu(x_ref[...], mu_ref, iv_ref, gm_ref, bt_ref)
    y = jax.lax.dot_general(W_ref[...], h, (((1,), (0,)), ((), ())),
                            preferred_element_type=jnp.float32) + b_ref[...]
    y_ref[...] = y
    c = y.shape[0]
    ys = y.reshape(c, y.shape[1] // 128, 128)
    s = jnp.sum(ys, axis=1)
    s2 = jnp.sum(ys * ys, axis=1)

    @pl.when(step == 0)
    def _init():
        acc_ref[0] = s
        acc_ref[1] = s2

    @pl.when(step > 0)
    def _acc():
        acc_ref[0] += s
        acc_ref[1] += s2

    @pl.when(step == nsteps - 1)
    def _emit():
        st_ref[...] = acc_ref[...]


def _bn_max_body(ns, mu_ref, iv_ref, gm_ref, bt_ref, x_ref, o_ref):
    h = _bn_relu(x_ref[...], mu_ref, iv_ref, gm_ref, bt_ref)
    c, t = h.shape
    o_ref[...] = jnp.max(h.reshape(c, t // ns, ns), axis=-1)


def _bn_relu_body(mu_ref, iv_ref, gm_ref, bt_ref, x_ref, o_ref):
    o_ref[...] = _bn_relu(x_ref[...], mu_ref, iv_ref, gm_ref, bt_ref)


def _col2(v):
    return v.reshape(-1, 1)


def _vec_spec(c):
    return pl.BlockSpec((c, 1), lambda t: (0, 0))


def _mm_stats(W, b, x):
    cout, cin = W.shape
    p = x.shape[1]
    tile = min(_TILE, p)
    nsteps = p // tile
    y, st = pl.pallas_call(
        functools.partial(_mm_stats_body, nsteps),
        grid=(nsteps,),
        in_specs=[pl.BlockSpec((cout, cin), lambda t: (0, 0)),
                  _vec_spec(cout),
                  pl.BlockSpec((cin, tile), lambda t: (0, t))],
        out_specs=[pl.BlockSpec((cout, tile), lambda t: (0, t)),
                   pl.BlockSpec((2, cout, 128), lambda t: (0, 0, 0))],
        out_shape=[jax.ShapeDtypeStruct((cout, p), jnp.float32),
                   jax.ShapeDtypeStruct((2, cout, 128), jnp.float32)],
        scratch_shapes=[pltpu.VMEM((2, cout, 128), jnp.float32)],
    )(W, _col2(b), x)
    return y, st


def _bn_mm_stats(mu, iv, gm, bt, W, b, x):
    cout, cin = W.shape
    p = x.shape[1]
    nsteps = p // _TILE
    y, st = pl.pallas_call(
        functools.partial(_bn_mm_stats_body, nsteps),
        grid=(nsteps,),
        in_specs=[_vec_spec(cin), _vec_spec(cin), _vec_spec(cin),
                  _vec_spec(cin),
                  pl.BlockSpec((cout, cin), lambda t: (0, 0)),
                  _vec_spec(cout),
                  pl.BlockSpec((cin, _TILE), lambda t: (0, t))],
        out_specs=[pl.BlockSpec((cout, _TILE), lambda t: (0, t)),
                   pl.BlockSpec((2, cout, 128), lambda t: (0, 0, 0))],
        out_shape=[jax.ShapeDtypeStruct((cout, p), jnp.float32),
                   jax.ShapeDtypeStruct((2, cout, 128), jnp.float32)],
        scratch_shapes=[pltpu.VMEM((2, cout, 128), jnp.float32)],
    )(_col2(mu), _col2(iv), _col2(gm), _col2(bt), W, _col2(b), x)
    return y, st


def _bn_max(mu, iv, gm, bt, x, ns):
    c, p = x.shape
    nsteps = p // _TILE
    return pl.pallas_call(
        functools.partial(_bn_max_body, ns),
        grid=(nsteps,),
        in_specs=[_vec_spec(c), _vec_spec(c), _vec_spec(c), _vec_spec(c),
                  pl.BlockSpec((c, _TILE), lambda t: (0, t))],
        out_specs=pl.BlockSpec((c, _TILE // ns), lambda t: (0, t)),
        out_shape=jax.ShapeDtypeStruct((c, p // ns), jnp.float32),
    )(_col2(mu), _col2(iv), _col2(gm), _col2(bt), x)


def _bn_relu_call(mu, iv, gm, bt, x):
    c, p = x.shape
    return pl.pallas_call(
        _bn_relu_body,
        grid=(1,),
        in_specs=[_vec_spec(c), _vec_spec(c), _vec_spec(c), _vec_spec(c),
                  pl.BlockSpec((c, p), lambda t: (0, 0))],
        out_specs=pl.BlockSpec((c, p), lambda t: (0, 0)),
        out_shape=jax.ShapeDtypeStruct((c, p), jnp.float32),
    )(_col2(mu), _col2(iv), _col2(gm), _col2(bt), x)


def _finalize_stats(st, p):
    s = st[0].sum(-1)
    s2 = st[1].sum(-1)
    mean = s / p
    var = s2 / p - mean * mean
    return mean, jax.lax.rsqrt(var + 1e-5)


def _square_distance(src, dst):
    return (jnp.sum(src ** 2, -1)[:, :, None] + jnp.sum(dst ** 2, -1)[:, None, :]
            - 2.0 * jnp.einsum('bmd,bnd->bmn', src, dst))


def _ball_query(radius, nsample, xyz, new_xyz):
    b, n, _ = xyz.shape
    m = new_xyz.shape[1]
    sqr = _square_distance(new_xyz, xyz)
    gidx = jnp.broadcast_to(jnp.arange(n, dtype=jnp.int32), (b, m, n))
    gidx = jnp.where(sqr > radius ** 2, n, gidx)
    gidx = jnp.sort(gidx, axis=-1)[:, :, :nsample]
    first = gidx[:, :, :1]
    first = jnp.where(first == n, 0, first)
    gidx = jnp.where(gidx == n, first, gidx)
    return gidx


def _gather_points(points, idx):
    bsz = points.shape[0]
    bidx = jnp.arange(bsz).reshape((bsz,) + (1,) * (idx.ndim - 1))
    return points[bidx, idx]


def _batchnorm(x, gamma, beta, axes):
    mean = jnp.mean(x, axis=axes, keepdims=True)
    var = jnp.var(x, axis=axes, keepdims=True)
    xh = (x - mean) * jax.lax.rsqrt(var + 1e-5)
    shape = [1] * x.ndim
    shape[1] = x.shape[1]
    return xh * gamma.reshape(shape) + beta.reshape(shape)


def kernel(xyz, feature, mlp_params, conv1_W, conv1_b, bn1_gamma, bn1_beta, fps_idx):
    inds, new_xyz = _run_fps(xyz)
    xs = _run_grouper(xyz, feature, new_xyz)
    outs = []
    for i in range(len(_RADII)):
        x = xs[i]
        p = x.shape[1]
        ns = _NSAMPLES[i]
        mu = iv = None
        for li, (W, bb, gm, bt) in enumerate(mlp_params[i]):
            if li == 0:
                x, st = _mm_stats(W, bb, x)
            else:
                x, st = _bn_mm_stats(mu, iv, gm_prev, bt_prev, W, bb, x)
            mu, iv = _finalize_stats(st, p)
            gm_prev, bt_prev = gm, bt
        outs.append(_bn_max(mu, iv, gm_prev, bt_prev, x, ns))
    nf_in = jnp.concatenate(outs, axis=0)
    y, st = _mm_stats(conv1_W, conv1_b, nf_in)
    mu, iv = _finalize_stats(st, _B * _NPOINTS)
    nf = _bn_relu_call(mu, iv, bn1_gamma, bn1_beta, y)
    nf = nf.reshape(_AGGC, _B, _NPOINTS).transpose(1, 0, 2)
    return new_xyz, nf, inds


# speedup vs baseline: 1.0608x; 1.0006x over previous
import functools

import jax
import jax.numpy as jnp
from jax import lax
from jax.experimental import pallas as pl
from jax.experimental.pallas import tpu as pltpu
from jax.experimental.pallas import tpu_sc as plsc

_B, _N = 2, 8192
_NPOINTS = 2048
_AGGC = 128
_RADII = [0.2, 0.4, 0.8]
_NSAMPLES = [16, 32, 64]
_SUBL, _LANE = 64, 128
_OSUB = 16


def _fps_body(xr, inds_ref, nxyz_ref):
    X = xr[0, 0]
    Y = xr[0, 1]
    Z = xr[0, 2]
    r_io = jax.lax.broadcasted_iota(jnp.int32, (_SUBL, _LANE), 0)
    c_io = jax.lax.broadcasted_iota(jnp.int32, (_SUBL, _LANE), 1)
    fi = r_io * _LANE + c_io
    r2 = jax.lax.broadcasted_iota(jnp.int32, (_OSUB, _LANE), 0)
    c2 = jax.lax.broadcasted_iota(jnp.int32, (_OSUB, _LANE), 1)
    fo = r2 * _LANE + c2
    BIG = jnp.int32(1 << 30)

    def step(i, st):
        dist, far, idxs, nx, ny, nz = st
        sel = fi == far
        cx = jnp.sum(jnp.where(sel, X, 0.0))
        cy = jnp.sum(jnp.where(sel, Y, 0.0))
        cz = jnp.sum(jnp.where(sel, Z, 0.0))
        dx = X - cx
        dy = Y - cy
        dz = Z - cz
        d = (dx * dx + dy * dy) + dz * dz
        dist = jnp.minimum(dist, d)
        m = jnp.max(dist)
        far_new = jnp.min(jnp.where(dist == m, fi, BIG))
        w = fo == i
        idxs = jnp.where(w, far, idxs)
        nx = jnp.where(w, cx, nx)
        ny = jnp.where(w, cy, ny)
        nz = jnp.where(w, cz, nz)
        return (dist, far_new, idxs, nx, ny, nz)

    dist0 = jnp.full((_SUBL, _LANE), 1e10, jnp.float32)
    zi = jnp.zeros((_OSUB, _LANE), jnp.int32)
    zf = jnp.zeros((_OSUB, _LANE), jnp.float32)
    _, _, idxs, nx, ny, nz = jax.lax.fori_loop(
        0, _NPOINTS, step, (dist0, jnp.int32(0), zi, zf, zf, zf))
    inds_ref[0] = idxs
    nxyz_ref[0, 0] = nx
    nxyz_ref[0, 1] = ny
    nxyz_ref[0, 2] = nz


def _run_fps(xyz):
    xr = xyz.transpose(0, 2, 1).reshape(_B, 3, _SUBL, _LANE)
    inds, nxyz = pl.pallas_call(
        _fps_body,
        grid=(_B,),
        in_specs=[pl.BlockSpec((1, 3, _SUBL, _LANE), lambda b: (b, 0, 0, 0))],
        out_specs=[
            pl.BlockSpec((1, _OSUB, _LANE), lambda b: (b, 0, 0)),
            pl.BlockSpec((1, 3, _OSUB, _LANE), lambda b: (b, 0, 0, 0)),
        ],
        out_shape=[
            jax.ShapeDtypeStruct((_B, _OSUB, _LANE), jnp.int32),
            jax.ShapeDtypeStruct((_B, 3, _OSUB, _LANE), jnp.float32),
        ],
    )(xr)
    inds = inds.reshape(_B, _NPOINTS)
    new_xyz = nxyz.reshape(_B, 3, _NPOINTS).transpose(0, 2, 1)
    return inds, new_xyz


def _sq_body(nx_ref, x_ref, o_ref):
    nx = nx_ref[0]
    x = x_ref[0]
    s1 = jnp.sum(nx ** 2, -1)[:, None]
    s2 = jnp.sum(x ** 2, -1)[None, :]
    dot = jax.lax.dot_general(nx, x, (((1,), (1,)), ((), ())),
                              preferred_element_type=jnp.float32)
    o_ref[0] = s1 + s2 - 2.0 * dot


def _run_sq(new_xyz, xyz):
    return pl.pallas_call(
        _sq_body,
        grid=(_B, 8),
        in_specs=[pl.BlockSpec((1, 256, 3), lambda b, m: (b, m, 0)),
                  pl.BlockSpec((1, _N, 3), lambda b, m: (b, 0, 0))],
        out_specs=pl.BlockSpec((1, 256, _N), lambda b, m: (b, m, 0)),
        out_shape=jax.ShapeDtypeStruct((_B, _NPOINTS, _N), jnp.float32),
    )(new_xyz, xyz)


_NCORE, _NSUB = 2, 16
_NW = _NCORE * _NSUB
_ROWS = _B * _NPOINTS
_RPW = _ROWS // _NW
_GRP = 16
_NGRP = _RPW // _GRP
_TAB_W = _N * 7
_TOT = [_ROWS * ns for ns in _NSAMPLES]
_RAD2 = [r * r for r in _RADII]


def _sc_body(tab_hbm, cen_hbm, sq_hbm, g1_hbm, g2_hbm, g3_hbm,
             tab_v, cen_v, sq_v, gb1, gb2, gb3, go1, go2, go3, cnt_s,
             sem0, sem1):
    wid = lax.axis_index("c") * _NSUB + lax.axis_index("s")
    base_row = wid * _RPW
    b = base_row // _NPOINTS
    pltpu.sync_copy(tab_hbm.at[pl.ds(b * _TAB_W, _TAB_W)], tab_v)
    pltpu.sync_copy(cen_hbm.at[pl.ds(base_row * 8, _RPW * 8)],
                    cen_v.at[pl.ds(0, _RPW * 8)])
    iota = lax.iota(jnp.int32, 16)
    zeros16 = jnp.zeros((16,), jnp.int32)
    gbs = (gb1, gb2, gb3)
    gos = (go1, go2, go3)
    ghs = (g1_hbm, g2_hbm, g3_hbm)
    sems = (sem0, sem1)
    sq_base = base_row * _N
    npairs = _RPW // 2

    def scan_row(rr, rloc, soff):
        cvec = cen_v[pl.ds(rloc * 8, 16)]
        cx = cvec[0]
        cy = cvec[1]
        cz = cvec[2]
        for k in range(3):
            gbs[k][pl.ds(0, 16)] = zeros16
            cnt_s[k] = jnp.int32(0)

        def chunk_body(j, carry3):
            c1 = cnt_s[0]
            c2 = cnt_s[1]
            c3 = cnt_s[2]
            live = (c1 < 16) | (c2 < 32) | (c3 < 64)

            @pl.when(live)
            def _do():
                idx = j * 16 + iota
                sq = sq_v[pl.ds(soff + j * 16, 16)]
                cs = [c1, c2, c3]
                for k in range(3):
                    mk = sq <= _RAD2[k]
                    csum = plsc.cumsum(mk.astype(jnp.int32))
                    pos = (cs[k] - 1) + csum
                    plsc.store_scatter(gbs[k], [pos], idx, mask=mk)
                    cnt_s[k] = jnp.minimum(
                        cs[k] + jnp.max(csum), _NSAMPLES[k])
            return carry3

        lax.fori_loop(0, _N // 16, chunk_body, 0)
        cnts = (cnt_s[0], cnt_s[1], cnt_s[2])
        for k in range(3):
            ns = _NSAMPLES[k]
            first = plsc.load_gather(gbs[k], [zeros16])
            for t in range(ns // 16):
                lane = t * 16 + iota
                v = gbs[k][pl.ds(t * 16, 16)]
                v = jnp.where(lane < cnts[k], v, first)
                v7 = v * 7
                colb = rr * ns + t * 16
                for c in range(7):
                    val = plsc.load_gather(tab_v, [v7 + c])
                    if c == 0:
                        val = val - cx
                    elif c == 1:
                        val = val - cy
                    elif c == 2:
                        val = val - cz
                    gos[k][pl.ds(c * (_GRP * ns) + colb, 16)] = val

    pltpu.async_copy(sq_hbm.at[pl.ds(sq_base, 2 * _N)],
                     sq_v.at[pl.ds(0, 2 * _N)], sem0)

    def group_body(gr, carry):
        def pair_body(pp, carry2):
            gp = gr * (_GRP // 2) + pp

            def do_pair(par):
                pltpu.make_async_copy(
                    sq_hbm.at[pl.ds(sq_base + gp * 2 * _N, 2 * _N)],
                    sq_v.at[pl.ds(par * 2 * _N, 2 * _N)],
                    sems[par]).wait()

                @pl.when(gp + 1 < npairs)
                def _pref():
                    pltpu.async_copy(
                        sq_hbm.at[pl.ds(sq_base + (gp + 1) * 2 * _N, 2 * _N)],
                        sq_v.at[pl.ds((1 - par) * 2 * _N, 2 * _N)],
                        sems[1 - par])
                for r01 in range(2):
                    rr = pp * 2 + r01
                    scan_row(rr, gr * _GRP + rr, (par * 2 + r01) * _N)

            @pl.when(pp % 2 == 0)
            def _even():
                do_pair(0)

            @pl.when(pp % 2 == 1)
            def _odd():
                do_pair(1)
            return carry2
        lax.fori_loop(0, _GRP // 2, pair_body, 0)
        for k in range(3):
            gsz = _GRP * _NSAMPLES[k]
            colbase = (base_row + gr * _GRP) * _NSAMPLES[k]
            for c in range(7):
                pltpu.sync_copy(
                    gos[k].at[pl.ds(c * gsz, gsz)],
                    ghs[k].at[pl.ds(c * _TOT[k] + colbase, gsz)])
        return carry
    lax.fori_loop(0, _NGRP, group_body, 0)


_sc_grouper = functools.partial(
    pl.kernel,
    mesh=plsc.VectorSubcoreMesh(core_axis_name="c", subcore_axis_name="s"),
    compiler_params=pltpu.CompilerParams(needs_layout_passes=False),
    out_type=[
        jax.ShapeDtypeStruct((7 * _TOT[0],), jnp.float32),
        jax.ShapeDtypeStruct((7 * _TOT[1],), jnp.float32),
        jax.ShapeDtypeStruct((7 * _TOT[2],), jnp.float32),
    ],
    scratch_types=[
        pltpu.VMEM((_TAB_W,), jnp.float32),
        pltpu.VMEM((_RPW * 8 + 8,), jnp.float32),
        pltpu.VMEM((4 * _N,), jnp.float32),
        pltpu.VMEM((16 + 16,), jnp.int32),
        pltpu.VMEM((32 + 16,), jnp.int32),
        pltpu.VMEM((64 + 16,), jnp.int32),
        pltpu.VMEM((7 * _GRP * 16,), jnp.float32),
        pltpu.VMEM((7 * _GRP * 32,), jnp.float32),
        pltpu.VMEM((7 * _GRP * 64,), jnp.float32),
        pltpu.SMEM((8,), jnp.int32),
        pltpu.SemaphoreType.DMA,
        pltpu.SemaphoreType.DMA,
    ],
)(_sc_body)


def _run_grouper(xyz, feature, new_xyz):
    feat_t = jnp.transpose(feature, (0, 2, 1))
    tab = jnp.concatenate([xyz, feat_t], axis=-1).reshape(-1)
    cen8 = jnp.zeros((_B * _NPOINTS, 8), jnp.float32)
    cen8 = cen8.at[:, :3].set(new_xyz.reshape(_B * _NPOINTS, 3))
    cen = cen8.reshape(-1)
    sq = _run_sq(new_xyz, xyz).reshape(-1)
    g1, g2, g3 = _sc_grouper(tab, cen, sq)
    return [g.reshape(7, tot) for g, tot in zip((g1, g2, g3), _TOT)]


_TILE = 8192


def _mm_stats_body(nsteps, W_ref, b_ref, x_ref, y_ref, st_ref, acc_ref):
    step = pl.program_id(0)
    y = jax.lax.dot_general(W_ref[...], x_ref[...], (((1,), (0,)), ((), ())),
                            preferred_element_type=jnp.float32) + b_ref[...]
    y_ref[...] = y
    c = y.shape[0]
    ys = y.reshape(c, y.shape[1] // 128, 128)
    s = jnp.sum(ys, axis=1)
    s2 = jnp.sum(ys * ys, axis=1)

    @pl.when(step == 0)
    def _init():
        acc_ref[0] = s
        acc_ref[1] = s2

    @pl.when(step > 0)
    def _acc():
        acc_ref[0] += s
        acc_ref[1] += s2

    @pl.when(step == nsteps - 1)
    def _emit():
        st_ref[...] = acc_ref[...]


def _bn_relu(y, mu_ref, iv_ref, gm_ref, bt_ref):
    xh = (y - mu_ref[...]) * iv_ref[...]
    return jax.nn.relu(xh * gm_ref[...] + bt_ref[...])


def _bn_mm_stats_body(nsteps, mu_ref, iv_ref, gm_ref, bt_ref, W_ref, b_ref,
                      x_ref, y_ref, st_ref, acc_ref):
    step = pl.program_id(0)
    h = _bn_relu(x_ref[...], mu_ref, iv_ref, gm_ref, bt_ref)
    y = jax.lax.dot_general(W_ref[...], h, (((1,), (0,)), ((), ())),
                            preferred_element_type=jnp.float32) + b_ref[...]
    y_ref[...] = y
    c = y.shape[0]
    ys = y.reshape(c, y.shape[1] // 128, 128)
    s = jnp.sum(ys, axis=1)
    s2 = jnp.sum(ys * ys, axis=1)

    @pl.when(step == 0)
    def _init():
        acc_ref[0] = s
        acc_ref[1] = s2

    @pl.when(step > 0)
    def _acc():
        acc_ref[0] += s
        acc_ref[1] += s2

    @pl.when(step == nsteps - 1)
    def _emit():
        st_ref[...] = acc_ref[...]


def _bn_max_body(ns, mu_ref, iv_ref, gm_ref, bt_ref, x_ref, o_ref):
    h = _bn_relu(x_ref[...], mu_ref, iv_ref, gm_ref, bt_ref)
    c, t = h.shape
    o_ref[...] = jnp.max(h.reshape(c, t // ns, ns), axis=-1)


def _bn_relu_body(mu_ref, iv_ref, gm_ref, bt_ref, x_ref, o_ref):
    o_ref[...] = _bn_relu(x_ref[...], mu_ref, iv_ref, gm_ref, bt_ref)


def _col2(v):
    return v.reshape(-1, 1)


def _vec_spec(c):
    return pl.BlockSpec((c, 1), lambda t: (0, 0))


def _mm_stats(W, b, x):
    cout, cin = W.shape
    p = x.shape[1]
    tile = min(_TILE, p)
    nsteps = p // tile
    y, st = pl.pallas_call(
        functools.partial(_mm_stats_body, nsteps),
        grid=(nsteps,),
        in_specs=[pl.BlockSpec((cout, cin), lambda t: (0, 0)),
                  _vec_spec(cout),
                  pl.BlockSpec((cin, tile), lambda t: (0, t))],
        out_specs=[pl.BlockSpec((cout, tile), lambda t: (0, t)),
                   pl.BlockSpec((2, cout, 128), lambda t: (0, 0, 0))],
        out_shape=[jax.ShapeDtypeStruct((cout, p), jnp.float32),
                   jax.ShapeDtypeStruct((2, cout, 128), jnp.float32)],
        scratch_shapes=[pltpu.VMEM((2, cout, 128), jnp.float32)],
    )(W, _col2(b), x)
    return y, st


def _bn_mm_stats(mu, iv, gm, bt, W, b, x):
    cout, cin = W.shape
    p = x.shape[1]
    nsteps = p // _TILE
    y, st = pl.pallas_call(
        functools.partial(_bn_mm_stats_body, nsteps),
        grid=(nsteps,),
        in_specs=[_vec_spec(cin), _vec_spec(cin), _vec_spec(cin),
                  _vec_spec(cin),
                  pl.BlockSpec((cout, cin), lambda t: (0, 0)),
                  _vec_spec(cout),
                  pl.BlockSpec((cin, _TILE), lambda t: (0, t))],
        out_specs=[pl.BlockSpec((cout, _TILE), lambda t: (0, t)),
                   pl.BlockSpec((2, cout, 128), lambda t: (0, 0, 0))],
        out_shape=[jax.ShapeDtypeStruct((cout, p), jnp.float32),
                   jax.ShapeDtypeStruct((2, cout, 128), jnp.float32)],
        scratch_shapes=[pltpu.VMEM((2, cout, 128), jnp.float32)],
    )(_col2(mu), _col2(iv), _col2(gm), _col2(bt), W, _col2(b), x)
    return y, st


def _bn_max(mu, iv, gm, bt, x, ns):
    c, p = x.shape
    nsteps = p // _TILE
    return pl.pallas_call(
        functools.partial(_bn_max_body, ns),
        grid=(nsteps,),
        in_specs=[_vec_spec(c), _vec_spec(c), _vec_spec(c), _vec_spec(c),
                  pl.BlockSpec((c, _TILE), lambda t: (0, t))],
        out_specs=pl.BlockSpec((c, _TILE // ns), lambda t: (0, t)),
        out_shape=jax.ShapeDtypeStruct((c, p // ns), jnp.float32),
    )(_col2(mu), _col2(iv), _col2(gm), _col2(bt), x)


def _bn_relu_call(mu, iv, gm, bt, x):
    c, p = x.shape
    return pl.pallas_call(
        _bn_relu_body,
        grid=(1,),
        in_specs=[_vec_spec(c), _vec_spec(c), _vec_spec(c), _vec_spec(c),
                  pl.BlockSpec((c, p), lambda t: (0, 0))],
        out_specs=pl.BlockSpec((c, p), lambda t: (0, 0)),
        out_shape=jax.ShapeDtypeStruct((c, p), jnp.float32),
    )(_col2(mu), _col2(iv), _col2(gm), _col2(bt), x)


def _finalize_stats(st, p):
    s = st[0].sum(-1)
    s2 = st[1].sum(-1)
    mean = s / p
    var = s2 / p - mean * mean
    return mean, jax.lax.rsqrt(var + 1e-5)


def kernel(xyz, feature, mlp_params, conv1_W, conv1_b, bn1_gamma, bn1_beta, fps_idx):
    inds, new_xyz = _run_fps(xyz)
    xs = _run_grouper(xyz, feature, new_xyz)
    outs = []
    for i in range(len(_RADII)):
        x = xs[i]
        p = x.shape[1]
        ns = _NSAMPLES[i]
        mu = iv = None
        for li, (W, bb, gm, bt) in enumerate(mlp_params[i]):
            if li == 0:
                x, st = _mm_stats(W, bb, x)
            else:
                x, st = _bn_mm_stats(mu, iv, gm_prev, bt_prev, W, bb, x)
            mu, iv = _finalize_stats(st, p)
            gm_prev, bt_prev = gm, bt
        outs.append(_bn_max(mu, iv, gm_prev, bt_prev, x, ns))
    nf_in = jnp.concatenate(outs, axis=0)
    y, st = _mm_stats(conv1_W, conv1_b, nf_in)
    mu, iv = _finalize_stats(st, _B * _NPOINTS)
    nf = _bn_relu_call(mu, iv, bn1_gamma, bn1_beta, y)
    nf = nf.reshape(_AGGC, _B, _NPOINTS).transpose(1, 0, 2)
    return new_xyz, nf, inds
